# Initial kernel scaffold; baseline (speedup 1.0000x reference)
import jax
import jax.numpy as jnp
from jax.experimental import pallas as pl


def kernel(edge_index, entity_embed0, entity_embed, user_embed):
    raise NotImplementedError("write your pallas kernel here")


# speedup vs baseline: 5.0062x; 5.0062x over previous
import functools

import jax
import jax.numpy as jnp
from jax import lax
from jax.experimental import pallas as pl
from jax.experimental.pallas import tpu as pltpu
from jax.experimental.pallas import tpu_sc as plsc

_N_USERS = 10000
_N_ITEMS = 10000
_DIM = 128
_N_EDGES = 320000
_K = 128
_NSUB = 16
_E_PAD = 327680
_CPT = _E_PAD // (_K * _NSUB)
_N_PAD = 10240
_RPT = _N_PAD // _NSUB
_ZCH = 128
_DEG_W = 16
_LANES = 16


def _sc_segment_sums(src1d, dst1d, item_tbl, item0_tbl):
    mesh = plsc.VectorSubcoreMesh(core_axis_name="c", subcore_axis_name="s")
    out_type = (
        jax.ShapeDtypeStruct((_N_PAD, _DIM), jnp.float32),
        jax.ShapeDtypeStruct((_N_PAD, _DIM), jnp.float32),
        jax.ShapeDtypeStruct((_N_PAD, _DEG_W), jnp.float32),
    )
    scratch = [
        pltpu.VMEM((_K,), jnp.int32),
        pltpu.VMEM((_K,), jnp.int32),
        pltpu.VMEM((_K, _DIM), jnp.float32),
        pltpu.VMEM((_K, _DEG_W), jnp.float32),
        pltpu.VMEM_SHARED((_N_PAD, _DIM), jnp.float32),
        pltpu.VMEM_SHARED((_N_PAD, _DEG_W), jnp.float32),
        pltpu.SemaphoreType.DMA,
    ]

    @functools.partial(pl.kernel, out_type=out_type, mesh=mesh,
                       scratch_types=scratch,
                       compiler_params=pltpu.CompilerParams(
                           use_tc_tiling_on_sc=False))
    def k(src_hbm, dst_hbm, item_hbm, item0_hbm, out1, out0, outdeg,
          src_v, dst_v, rows, ones, acc, dacc, sem):
        c = lax.axis_index("c")
        s = lax.axis_index("s")

        @pl.loop(0, _K)
        def _(i):
            ones[i, pl.ds(0, _LANES)] = jnp.zeros((_LANES,), jnp.float32)

            @pl.loop(0, _DIM // _LANES)
            def _(j):
                rows[i, pl.ds(j * _LANES, _LANES)] = jnp.zeros(
                    (_LANES,), jnp.float32)

        @pl.loop(0, _RPT // _ZCH)
        def _(b):
            base = s * _RPT + b * _ZCH
            pltpu.sync_copy(rows, acc.at[pl.ds(base, _ZCH)])
            pltpu.sync_copy(ones, dacc.at[pl.ds(base, _ZCH)])

        @pl.loop(0, _K)
        def _(i):
            ones[i, pl.ds(0, _LANES)] = jnp.ones((_LANES,), jnp.float32)

        plsc.subcore_barrier()

        @pl.loop(0, _CPT)
        def _(j):
            base = (s * _CPT + j) * _K
            pltpu.sync_copy(src_hbm.at[pl.ds(base, _K)], src_v)
            pltpu.sync_copy(dst_hbm.at[pl.ds(base, _K)], dst_v)

            @pl.when(c == 0)
            def _():
                pltpu.async_copy(item_hbm.at[src_v], rows, sem).wait()
                pltpu.sync_copy(rows, acc.at[dst_v], add=True)
                pltpu.sync_copy(ones, dacc.at[dst_v], add=True)

            @pl.when(c == 1)
            def _():
                pltpu.async_copy(item0_hbm.at[src_v], rows, sem).wait()
                pltpu.sync_copy(rows, acc.at[dst_v], add=True)

        plsc.subcore_barrier()

        @pl.loop(0, _RPT // _ZCH)
        def _(b):
            base = s * _RPT + b * _ZCH
            sl = pl.ds(base, _ZCH)

            @pl.when(c == 0)
            def _():
                pltpu.sync_copy(acc.at[sl], rows)
                pltpu.sync_copy(rows, out1.at[sl])
                pltpu.sync_copy(dacc.at[sl], ones)
                pltpu.sync_copy(ones, outdeg.at[sl])

            @pl.when(c == 1)
            def _():
                pltpu.sync_copy(acc.at[sl], rows)
                pltpu.sync_copy(rows, out0.at[sl])

    return k(src1d, dst1d, item_tbl, item0_tbl)


_DIM_FLAG1 = (0, 32, 64, 96, 128)
_K_ATT = 1.0


def _tc_finish_body(sum1, sum0, dg, ue, it, out):
    d = jnp.maximum(dg[:, 0:1], 1.0)
    emb_f = sum1[...] / d
    emb_f0 = sum0[...] / d
    colmean = jnp.mean(it[...], axis=0, keepdims=True)
    u = ue[...]
    for i in range(4):
        lo, hi = _DIM_FLAG1[i], _DIM_FLAG1[i + 1]
        user_att = jnp.sum(emb_f[:, lo:hi] * u[:, lo:hi], axis=1,
                           keepdims=True)
        user_att = jax.nn.relu(user_att) + 1e-10
        mean_att = jnp.sum(colmean[:, lo:hi] * u[:, lo:hi], axis=1,
                           keepdims=True)
        mean_att = jax.nn.relu(mean_att) + 1e-08
        att = _K_ATT * jax.nn.relu(user_att / mean_att - 1.0) + 0.01
        score = jnp.tanh(att)
        out[:, lo:hi] = score * emb_f[:, lo:hi] + emb_f0[:, lo:hi]


def _tc_finish(sum1, sum0, deg, user_embed, entity_embed):
    return pl.pallas_call(
        _tc_finish_body,
        grid=(1,),
        out_shape=jax.ShapeDtypeStruct((_N_USERS, _DIM), jnp.float32),
        in_specs=[
            pl.BlockSpec((_N_USERS, _DIM), lambda i: (0, 0)),
            pl.BlockSpec((_N_USERS, _DIM), lambda i: (0, 0)),
            pl.BlockSpec((_N_USERS, _DEG_W), lambda i: (0, 0)),
            pl.BlockSpec((_N_USERS, _DIM), lambda i: (0, 0)),
            pl.BlockSpec((_N_ITEMS, _DIM), lambda i: (0, 0)),
        ],
        out_specs=pl.BlockSpec((_N_USERS, _DIM), lambda i: (0, 0)),
    )(sum1, sum0, deg, user_embed, entity_embed)


def kernel(edge_index, entity_embed0, entity_embed, user_embed):
    n_extra = _E_PAD - _N_EDGES
    pad_src = jnp.arange(n_extra, dtype=jnp.int32) % _N_ITEMS
    pad_dst = _N_USERS + (jnp.arange(n_extra, dtype=jnp.int32)
                          % (_N_PAD - _N_USERS))
    src1d = jnp.concatenate([edge_index[0], pad_src])
    dst1d = jnp.concatenate([edge_index[1], pad_dst])
    sum1, sum0, deg = _sc_segment_sums(src1d, dst1d, entity_embed,
                                       entity_embed0)
    return _tc_finish(sum1, sum0, deg, user_embed, entity_embed)


# speedup vs baseline: 8.8660x; 1.7710x over previous
import functools

import jax
import jax.numpy as jnp
from jax import lax
from jax.experimental import pallas as pl
from jax.experimental.pallas import tpu as pltpu
from jax.experimental.pallas import tpu_sc as plsc

_N_USERS = 10000
_N_ITEMS = 10000
_DIM = 128
_N_EDGES = 320000
_K = 128
_NSUB = 16
_E_PAD = 327680
_CPT = _E_PAD // (_K * _NSUB)
_N_PAD = 10240
_RPT = _N_PAD // _NSUB
_ZCH = 128
_DEG_W = 16
_LANES = 16
_IDXB = 8
_NBLK = _CPT // _IDXB
_NCHUNK = _E_PAD // _K


def _sc_segment_sums(src_hbm2d, dst_hbm2d, item_tbl, item0_tbl):
    mesh = plsc.VectorSubcoreMesh(core_axis_name="c", subcore_axis_name="s")
    out_type = (
        jax.ShapeDtypeStruct((_N_PAD, _DIM), jnp.float32),
        jax.ShapeDtypeStruct((_N_PAD, _DIM), jnp.float32),
        jax.ShapeDtypeStruct((_N_PAD, _DEG_W), jnp.float32),
    )
    scratch = [
        pltpu.VMEM((_IDXB, _K), jnp.int32),
        pltpu.VMEM((_IDXB, _K), jnp.int32),
        pltpu.VMEM((_K, _DIM), jnp.float32),
        pltpu.VMEM((_K, _DIM), jnp.float32),
        pltpu.VMEM((_K, _DEG_W), jnp.float32),
        pltpu.VMEM_SHARED((_N_PAD, _DIM), jnp.float32),
        pltpu.VMEM_SHARED((_N_PAD, _DEG_W), jnp.float32),
        pltpu.SemaphoreType.DMA,
        pltpu.SemaphoreType.DMA,
        pltpu.SemaphoreType.DMA,
        pltpu.SemaphoreType.DMA,
    ]

    @functools.partial(pl.kernel, out_type=out_type, mesh=mesh,
                       scratch_types=scratch,
                       compiler_params=pltpu.CompilerParams(
                           use_tc_tiling_on_sc=False))
    def k(src_hbm, dst_hbm, item_hbm, item0_hbm, out1, out0, outdeg,
          src_v, dst_v, rows_a, rows_b, ones, acc, dacc,
          g_a, g_b, s_a, s_b):
        c = lax.axis_index("c")
        s = lax.axis_index("s")

        @pl.loop(0, _K)
        def _(i):
            ones[i, pl.ds(0, _LANES)] = jnp.zeros((_LANES,), jnp.float32)

            @pl.loop(0, _DIM // _LANES)
            def _(j):
                rows_a[i, pl.ds(j * _LANES, _LANES)] = jnp.zeros(
                    (_LANES,), jnp.float32)

        @pl.loop(0, _RPT // _ZCH)
        def _(b):
            base = s * _RPT + b * _ZCH
            pltpu.sync_copy(rows_a, acc.at[pl.ds(base, _ZCH)])
            pltpu.sync_copy(ones, dacc.at[pl.ds(base, _ZCH)])

        @pl.loop(0, _K)
        def _(i):
            ones[i, pl.ds(0, _LANES)] = jnp.ones((_LANES,), jnp.float32)

        plsc.subcore_barrier()

        def fire_gather(i, buf, gsem):
            idx = src_v.at[i]

            @pl.when(c == 0)
            def _():
                pltpu.async_copy(item_hbm.at[idx], buf, gsem)

            @pl.when(c == 1)
            def _():
                pltpu.async_copy(item0_hbm.at[idx], buf, gsem)

        def wait_gather(buf, gsem):
            pltpu.make_async_copy(item_hbm.at[src_v.at[0]], buf,
                                  gsem).wait()

        def fire_scatter(i, buf, ssem):
            pltpu.async_copy(buf, acc.at[dst_v.at[i]], ssem, add=True)

            @pl.when(c == 0)
            def _():
                pltpu.async_copy(ones, dacc.at[dst_v.at[i]], ssem,
                                 add=True)

        def wait_scatter(buf, ssem):
            pltpu.make_async_copy(buf, acc.at[dst_v.at[0]], ssem).wait()

            @pl.when(c == 0)
            def _():
                pltpu.make_async_copy(ones, dacc.at[dst_v.at[0]],
                                      ssem).wait()

        bufs = ((rows_a, g_a, s_a), (rows_b, g_b, s_b))

        @pl.loop(0, _NBLK)
        def _(b):
            cbase = s * _CPT + b * _IDXB
            pltpu.sync_copy(src_hbm.at[pl.ds(cbase, _IDXB)], src_v)
            pltpu.sync_copy(dst_hbm.at[pl.ds(cbase, _IDXB)], dst_v)

            fire_gather(0, rows_a, g_a)
            for i in range(_IDXB):
                cur_buf, cur_g, cur_s = bufs[i % 2]
                nxt_buf, nxt_g, nxt_s = bufs[(i + 1) % 2]
                if i + 1 < _IDXB:
                    if i >= 1:
                        wait_scatter(nxt_buf, nxt_s)
                    fire_gather(i + 1, nxt_buf, nxt_g)
                wait_gather(cur_buf, cur_g)
                fire_scatter(i, cur_buf, cur_s)
            wait_scatter(rows_a, s_a)
            wait_scatter(rows_b, s_b)

        plsc.subcore_barrier()

        @pl.loop(0, _RPT // _ZCH)
        def _(b):
            base = s * _RPT + b * _ZCH
            sl = pl.ds(base, _ZCH)

            @pl.when(c == 0)
            def _():
                pltpu.sync_copy(acc.at[sl], rows_a)
                pltpu.sync_copy(rows_a, out1.at[sl])
                pltpu.sync_copy(dacc.at[sl], ones)
                pltpu.sync_copy(ones, outdeg.at[sl])

            @pl.when(c == 1)
            def _():
                pltpu.sync_copy(acc.at[sl], rows_a)
                pltpu.sync_copy(rows_a, out0.at[sl])

    return k(src_hbm2d, dst_hbm2d, item_tbl, item0_tbl)


_DIM_FLAG1 = (0, 32, 64, 96, 128)
_K_ATT = 1.0


def _tc_finish_body(sum1, sum0, dg, ue, it, out):
    d = jnp.maximum(dg[:, 0:1], 1.0)
    emb_f = sum1[...] / d
    emb_f0 = sum0[...] / d
    colmean = jnp.mean(it[...], axis=0, keepdims=True)
    u = ue[...]
    for i in range(4):
        lo, hi = _DIM_FLAG1[i], _DIM_FLAG1[i + 1]
        user_att = jnp.sum(emb_f[:, lo:hi] * u[:, lo:hi], axis=1,
                           keepdims=True)
        user_att = jax.nn.relu(user_att) + 1e-10
        mean_att = jnp.sum(colmean[:, lo:hi] * u[:, lo:hi], axis=1,
                           keepdims=True)
        mean_att = jax.nn.relu(mean_att) + 1e-08
        att = _K_ATT * jax.nn.relu(user_att / mean_att - 1.0) + 0.01
        score = jnp.tanh(att)
        out[:, lo:hi] = score * emb_f[:, lo:hi] + emb_f0[:, lo:hi]


def _tc_finish(sum1, sum0, deg, user_embed, entity_embed):
    return pl.pallas_call(
        _tc_finish_body,
        grid=(1,),
        out_shape=jax.ShapeDtypeStruct((_N_USERS, _DIM), jnp.float32),
        in_specs=[
            pl.BlockSpec((_N_USERS, _DIM), lambda i: (0, 0)),
            pl.BlockSpec((_N_USERS, _DIM), lambda i: (0, 0)),
            pl.BlockSpec((_N_USERS, _DEG_W), lambda i: (0, 0)),
            pl.BlockSpec((_N_USERS, _DIM), lambda i: (0, 0)),
            pl.BlockSpec((_N_ITEMS, _DIM), lambda i: (0, 0)),
        ],
        out_specs=pl.BlockSpec((_N_USERS, _DIM), lambda i: (0, 0)),
    )(sum1, sum0, deg, user_embed, entity_embed)


def kernel(edge_index, entity_embed0, entity_embed, user_embed):
    n_extra = _E_PAD - _N_EDGES
    pad_src = jnp.arange(n_extra, dtype=jnp.int32) % _N_ITEMS
    pad_dst = _N_USERS + (jnp.arange(n_extra, dtype=jnp.int32)
                          % (_N_PAD - _N_USERS))
    src2d = jnp.concatenate([edge_index[0], pad_src]).reshape(_NCHUNK, _K)
    dst2d = jnp.concatenate([edge_index[1], pad_dst]).reshape(_NCHUNK, _K)
    sum1, sum0, deg = _sc_segment_sums(src2d, dst2d, entity_embed,
                                       entity_embed0)
    return _tc_finish(sum1, sum0, deg, user_embed, entity_embed)


# speedup vs baseline: 9.8284x; 1.1086x over previous
import functools

import jax
import jax.numpy as jnp
from jax import lax
from jax.experimental import pallas as pl
from jax.experimental.pallas import tpu as pltpu
from jax.experimental.pallas import tpu_sc as plsc

_N_USERS = 10000
_N_ITEMS = 10000
_DIM = 128
_N_EDGES = 320000
_K = 128
_NSUB = 16
_E_PAD = 327680
_CPT = _E_PAD // (_K * _NSUB)
_N_PAD = 10240
_RPT = _N_PAD // _NSUB
_ZCH = 128
_DEG_W = 8
_LANES = 16
_IDXB = 16
_NBLK = _CPT // _IDXB
_NCHUNK = _E_PAD // _K


def _sc_segment_sums(src_hbm2d, dst_hbm2d, item_tbl, item0_tbl):
    mesh = plsc.VectorSubcoreMesh(core_axis_name="c", subcore_axis_name="s")
    out_type = (
        jax.ShapeDtypeStruct((_N_PAD, _DIM), jnp.float32),
        jax.ShapeDtypeStruct((_N_PAD, _DIM), jnp.float32),
        jax.ShapeDtypeStruct((_N_PAD, _DEG_W), jnp.float32),
        jax.ShapeDtypeStruct((_N_PAD, _DEG_W), jnp.float32),
    )
    scratch = [
        pltpu.VMEM((_IDXB, _K), jnp.int32),
        pltpu.VMEM((_IDXB, _K), jnp.int32),
        pltpu.VMEM((_IDXB, _K), jnp.int32),
        pltpu.VMEM((_IDXB, _K), jnp.int32),
        pltpu.VMEM((_K, _DIM), jnp.float32),
        pltpu.VMEM((_K, _DIM), jnp.float32),
        pltpu.VMEM((_K, _DEG_W), jnp.float32),
        pltpu.VMEM_SHARED((_N_PAD, _DIM), jnp.float32),
        pltpu.VMEM_SHARED((_N_PAD, _DEG_W), jnp.float32),
        pltpu.SemaphoreType.DMA,
        pltpu.SemaphoreType.DMA,
        pltpu.SemaphoreType.DMA,
        pltpu.SemaphoreType.DMA,
        pltpu.SemaphoreType.DMA,
        pltpu.SemaphoreType.DMA,
    ]

    @functools.partial(pl.kernel, out_type=out_type, mesh=mesh,
                       scratch_types=scratch,
                       compiler_params=pltpu.CompilerParams(
                           use_tc_tiling_on_sc=False))
    def k(src_hbm, dst_hbm, item_hbm, item0_hbm, out1, out0, outdeg_a,
          outdeg_b, src_a, dst_a, src_b, dst_b, rows_a, rows_b, ones,
          acc, dacc, g_a, g_b, s_a, s_b, i_a, i_b):
        c = lax.axis_index("c")
        s = lax.axis_index("s")

        @pl.loop(0, _K)
        def _(i):
            @pl.loop(0, _DIM // _LANES)
            def _(j):
                rows_a[i, pl.ds(j * _LANES, _LANES)] = jnp.zeros(
                    (_LANES,), jnp.float32)

        @pl.loop(0, _RPT // _ZCH)
        def _(b):
            base = s * _RPT + b * _ZCH
            pltpu.sync_copy(rows_a, acc.at[pl.ds(base, _ZCH)])
            pltpu.sync_copy(rows_a.at[pl.ds(0, _ZCH), pl.ds(0, _DEG_W)],
                            dacc.at[pl.ds(base, _ZCH)])

        @pl.loop(0, _K)
        def _(i):
            rows_a[i, pl.ds(0, _LANES)] = jnp.ones((_LANES,), jnp.float32)

        plsc.subcore_barrier()
        pad_sl = pl.ds(_N_PAD - _K, _K)
        pltpu.sync_copy(rows_a.at[pl.ds(0, _K), pl.ds(0, _DEG_W)],
                        dacc.at[pad_sl])
        pltpu.sync_copy(dacc.at[pad_sl], ones)

        plsc.subcore_barrier()

        def fire_gather(sv, i, buf, gsem):
            idx = sv.at[i]

            @pl.when(c == 0)
            def _():
                pltpu.async_copy(item_hbm.at[idx], buf, gsem)

            @pl.when(c == 1)
            def _():
                pltpu.async_copy(item0_hbm.at[idx], buf, gsem)

        def wait_gather(sv, buf, gsem):
            pltpu.make_async_copy(item_hbm.at[sv.at[0]], buf, gsem).wait()

        def fire_scatter(dv, i, buf, ssem, pdeg):
            pltpu.async_copy(buf, acc.at[dv.at[i]], ssem, add=True)

            @pl.when(pdeg)
            def _():
                pltpu.async_copy(ones, dacc.at[dv.at[i]], ssem, add=True)

        def wait_scatter(dv, buf, ssem, pdeg):
            pltpu.make_async_copy(buf, acc.at[dv.at[0]], ssem).wait()

            @pl.when(pdeg)
            def _():
                pltpu.make_async_copy(ones, dacc.at[dv.at[0]],
                                      ssem).wait()

        def stage_idx(b, sbuf, dbuf, isem):
            cbase = s * _CPT + b * _IDXB
            pltpu.async_copy(src_hbm.at[pl.ds(cbase, _IDXB)], sbuf, isem)
            pltpu.async_copy(dst_hbm.at[pl.ds(cbase, _IDXB)], dbuf, isem)

        def wait_idx(sbuf, dbuf, isem):
            pltpu.make_async_copy(src_hbm.at[pl.ds(0, _IDXB)], sbuf,
                                  isem).wait()
            pltpu.make_async_copy(dst_hbm.at[pl.ds(0, _IDXB)], dbuf,
                                  isem).wait()

        bufs = ((rows_a, g_a, s_a), (rows_b, g_b, s_b))
        half = _NBLK // 2

        def process_block(b, sv, dv):
            pdeg = jnp.logical_or(
                jnp.logical_and(c == 0, b < half),
                jnp.logical_and(c == 1, b >= half))
            fire_gather(sv, 0, rows_a, g_a)
            for i in range(_IDXB):
                cur_buf, cur_g, cur_s = bufs[i % 2]
                nxt_buf, nxt_g, nxt_s = bufs[(i + 1) % 2]
                if i + 1 < _IDXB:
                    if i >= 1:
                        wait_scatter(dv, nxt_buf, nxt_s, pdeg)
                    fire_gather(sv, i + 1, nxt_buf, nxt_g)
                wait_gather(sv, cur_buf, cur_g)
                fire_scatter(dv, i, cur_buf, cur_s, pdeg)
            wait_scatter(dv, rows_a, s_a, pdeg)
            wait_scatter(dv, rows_b, s_b, pdeg)

        stage_idx(0, src_a, dst_a, i_a)

        @pl.loop(0, _NBLK // 2)
        def _(p):
            b0 = 2 * p
            wait_idx(src_a, dst_a, i_a)
            stage_idx(b0 + 1, src_b, dst_b, i_b)
            process_block(b0, src_a, dst_a)
            wait_idx(src_b, dst_b, i_b)

            @pl.when(p + 1 < _NBLK // 2)
            def _():
                stage_idx(b0 + 2, src_a, dst_a, i_a)

            process_block(b0 + 1, src_b, dst_b)

        plsc.subcore_barrier()

        @pl.loop(0, _RPT // _ZCH)
        def _(b):
            base = s * _RPT + b * _ZCH
            sl = pl.ds(base, _ZCH)
            pltpu.sync_copy(acc.at[sl], rows_a)
            pltpu.sync_copy(dacc.at[sl], ones)

            @pl.when(c == 0)
            def _():
                pltpu.sync_copy(rows_a, out1.at[sl])
                pltpu.sync_copy(ones, outdeg_a.at[sl])

            @pl.when(c == 1)
            def _():
                pltpu.sync_copy(rows_a, out0.at[sl])
                pltpu.sync_copy(ones, outdeg_b.at[sl])

    return k(src_hbm2d, dst_hbm2d, item_tbl, item0_tbl)


_DIM_FLAG1 = (0, 32, 64, 96, 128)
_K_ATT = 1.0


def _tc_finish_body(sum1, sum0, dg, ue, it, out):
    d = jnp.maximum(dg[:, 0:1], 1.0)
    emb_f = sum1[...] / d
    emb_f0 = sum0[...] / d
    colmean = jnp.mean(it[...], axis=0, keepdims=True)
    u = ue[...]
    for i in range(4):
        lo, hi = _DIM_FLAG1[i], _DIM_FLAG1[i + 1]
        user_att = jnp.sum(emb_f[:, lo:hi] * u[:, lo:hi], axis=1,
                           keepdims=True)
        user_att = jax.nn.relu(user_att) + 1e-10
        mean_att = jnp.sum(colmean[:, lo:hi] * u[:, lo:hi], axis=1,
                           keepdims=True)
        mean_att = jax.nn.relu(mean_att) + 1e-08
        att = _K_ATT * jax.nn.relu(user_att / mean_att - 1.0) + 0.01
        score = jnp.tanh(att)
        out[:, lo:hi] = score * emb_f[:, lo:hi] + emb_f0[:, lo:hi]


def _tc_finish(sum1, sum0, deg, user_embed, entity_embed):
    return pl.pallas_call(
        _tc_finish_body,
        grid=(1,),
        out_shape=jax.ShapeDtypeStruct((_N_USERS, _DIM), jnp.float32),
        in_specs=[
            pl.BlockSpec((_N_USERS, _DIM), lambda i: (0, 0)),
            pl.BlockSpec((_N_USERS, _DIM), lambda i: (0, 0)),
            pl.BlockSpec((_N_USERS, _DEG_W), lambda i: (0, 0)),
            pl.BlockSpec((_N_USERS, _DIM), lambda i: (0, 0)),
            pl.BlockSpec((_N_ITEMS, _DIM), lambda i: (0, 0)),
        ],
        out_specs=pl.BlockSpec((_N_USERS, _DIM), lambda i: (0, 0)),
    )(sum1, sum0, deg, user_embed, entity_embed)


def kernel(edge_index, entity_embed0, entity_embed, user_embed):
    n_extra = _E_PAD - _N_EDGES
    pad_src = jnp.arange(n_extra, dtype=jnp.int32)
    pad_dst = _N_USERS + (jnp.arange(n_extra, dtype=jnp.int32) & 63)
    src2d = jnp.concatenate([edge_index[0], pad_src]).reshape(_NCHUNK, _K)
    dst2d = jnp.concatenate([edge_index[1], pad_dst]).reshape(_NCHUNK, _K)
    sum1, sum0, deg_a, deg_b = _sc_segment_sums(src2d, dst2d, entity_embed,
                                                entity_embed0)
    return _tc_finish(sum1, sum0, deg_a + deg_b, user_embed, entity_embed)


# speedup vs baseline: 10.1852x; 1.0363x over previous
import functools

import jax
import jax.numpy as jnp
from jax import lax
from jax.experimental import pallas as pl
from jax.experimental.pallas import tpu as pltpu
from jax.experimental.pallas import tpu_sc as plsc

_N_USERS = 10000
_N_ITEMS = 10000
_DIM = 128
_N_EDGES = 320000
_K = 128
_NSUB = 16
_E_PAD = 327680
_CPT = _E_PAD // (_K * _NSUB)
_N_PAD = 10240
_RPT = _N_PAD // _NSUB
_ZCH = 128
_DEG_W = 8
_LANES = 16
_IDXB = 16
_NBLK = _CPT // _IDXB
_NCHUNK = _E_PAD // _K


def _sc_segment_sums(src_hbm2d, dst_hbm2d, item_tbl, item0_tbl):
    mesh = plsc.VectorSubcoreMesh(core_axis_name="c", subcore_axis_name="s")
    out_type = (
        jax.ShapeDtypeStruct((_N_PAD, _DIM), jnp.float32),
        jax.ShapeDtypeStruct((_N_PAD, _DIM), jnp.float32),
        jax.ShapeDtypeStruct((_N_PAD, _DEG_W), jnp.float32),
        jax.ShapeDtypeStruct((_N_PAD, _DEG_W), jnp.float32),
    )
    scratch = [
        pltpu.VMEM((_IDXB, _K), jnp.int32),
        pltpu.VMEM((_IDXB, _K), jnp.int32),
        pltpu.VMEM((_IDXB, _K), jnp.int32),
        pltpu.VMEM((_IDXB, _K), jnp.int32),
        pltpu.VMEM((_K, _DIM), jnp.float32),
        pltpu.VMEM((_K, _DIM), jnp.float32),
        pltpu.VMEM((_K, _DEG_W), jnp.float32),
        pltpu.VMEM_SHARED((_N_PAD, _DIM), jnp.float32),
        pltpu.VMEM_SHARED((_N_PAD, _DEG_W), jnp.float32),
        pltpu.SemaphoreType.DMA,
        pltpu.SemaphoreType.DMA,
        pltpu.SemaphoreType.DMA,
        pltpu.SemaphoreType.DMA,
        pltpu.SemaphoreType.DMA,
        pltpu.SemaphoreType.DMA,
    ]

    @functools.partial(pl.kernel, out_type=out_type, mesh=mesh,
                       scratch_types=scratch,
                       compiler_params=pltpu.CompilerParams(
                           use_tc_tiling_on_sc=False))
    def k(src_hbm, dst_hbm, item_hbm, item0_hbm, out1, out0, outdeg_a,
          outdeg_b, src_a, dst_a, src_b, dst_b, rows_a, rows_b, ones,
          acc, dacc, g_a, g_b, s_a, s_b, i_a, i_b):
        c = lax.axis_index("c")
        s = lax.axis_index("s")

        @pl.loop(0, _K)
        def _(i):
            @pl.loop(0, _DIM // _LANES)
            def _(j):
                rows_a[i, pl.ds(j * _LANES, _LANES)] = jnp.zeros(
                    (_LANES,), jnp.float32)

        @pl.loop(0, _RPT // _ZCH)
        def _(b):
            base = s * _RPT + b * _ZCH
            pltpu.sync_copy(rows_a, acc.at[pl.ds(base, _ZCH)])
            pltpu.sync_copy(rows_a.at[pl.ds(0, _ZCH), pl.ds(0, _DEG_W)],
                            dacc.at[pl.ds(base, _ZCH)])

        @pl.loop(0, _K)
        def _(i):
            rows_a[i, pl.ds(0, _LANES)] = jnp.ones((_LANES,), jnp.float32)

        plsc.subcore_barrier()
        pad_sl = pl.ds(_N_PAD - _K, _K)
        pltpu.sync_copy(rows_a.at[pl.ds(0, _K), pl.ds(0, _DEG_W)],
                        dacc.at[pad_sl])
        pltpu.sync_copy(dacc.at[pad_sl], ones)

        plsc.subcore_barrier()

        def fire_gather(sv, i, buf, gsem):
            idx = sv.at[i]

            @pl.when(c == 0)
            def _():
                pltpu.async_copy(item_hbm.at[idx], buf, gsem)

            @pl.when(c == 1)
            def _():
                pltpu.async_copy(item0_hbm.at[idx], buf, gsem)

        def wait_gather(sv, buf, gsem):
            pltpu.make_async_copy(item_hbm.at[sv.at[0]], buf, gsem).wait()

        def fire_scatter(dv, i, buf, ssem, pdeg):
            pltpu.async_copy(buf, acc.at[dv.at[i]], ssem, add=True)

            @pl.when(pdeg)
            def _():
                pltpu.async_copy(ones, dacc.at[dv.at[i]], ssem, add=True)

        def wait_scatter(dv, buf, ssem, pdeg):
            pltpu.make_async_copy(buf, acc.at[dv.at[0]], ssem).wait()

            @pl.when(pdeg)
            def _():
                pltpu.make_async_copy(ones, dacc.at[dv.at[0]],
                                      ssem).wait()

        def stage_idx(b, sbuf, dbuf, isem):
            cbase = s * _CPT + b * _IDXB
            pltpu.async_copy(src_hbm.at[pl.ds(cbase, _IDXB)], sbuf, isem)
            pltpu.async_copy(dst_hbm.at[pl.ds(cbase, _IDXB)], dbuf, isem)

        def wait_idx(sbuf, dbuf, isem):
            pltpu.make_async_copy(src_hbm.at[pl.ds(0, _IDXB)], sbuf,
                                  isem).wait()
            pltpu.make_async_copy(dst_hbm.at[pl.ds(0, _IDXB)], dbuf,
                                  isem).wait()

        bufs = ((rows_a, g_a, s_a), (rows_b, g_b, s_b))
        half = _NBLK // 2

        def process_block(b, sv, dv):
            pdeg = jnp.logical_or(
                jnp.logical_and(c == 0, b < half),
                jnp.logical_and(c == 1, b >= half))
            fire_gather(sv, 0, rows_a, g_a)
            for i in range(_IDXB):
                cur_buf, cur_g, cur_s = bufs[i % 2]
                nxt_buf, nxt_g, nxt_s = bufs[(i + 1) % 2]
                if i + 1 < _IDXB:
                    if i >= 1:
                        wait_scatter(dv, nxt_buf, nxt_s, pdeg)
                    fire_gather(sv, i + 1, nxt_buf, nxt_g)
                wait_gather(sv, cur_buf, cur_g)
                fire_scatter(dv, i, cur_buf, cur_s, pdeg)
            wait_scatter(dv, rows_a, s_a, pdeg)
            wait_scatter(dv, rows_b, s_b, pdeg)

        stage_idx(0, src_a, dst_a, i_a)

        @pl.loop(0, _NBLK // 2)
        def _(p):
            b0 = 2 * p
            wait_idx(src_a, dst_a, i_a)
            stage_idx(b0 + 1, src_b, dst_b, i_b)
            process_block(b0, src_a, dst_a)
            wait_idx(src_b, dst_b, i_b)

            @pl.when(p + 1 < _NBLK // 2)
            def _():
                stage_idx(b0 + 2, src_a, dst_a, i_a)

            process_block(b0 + 1, src_b, dst_b)

        plsc.subcore_barrier()

        @pl.loop(0, _RPT // _ZCH)
        def _(b):
            base = s * _RPT + b * _ZCH
            sl = pl.ds(base, _ZCH)
            pltpu.sync_copy(acc.at[sl], rows_a)
            pltpu.sync_copy(dacc.at[sl], ones)

            @pl.when(c == 0)
            def _():
                pltpu.sync_copy(rows_a, out1.at[sl])
                pltpu.sync_copy(ones, outdeg_a.at[sl])

            @pl.when(c == 1)
            def _():
                pltpu.sync_copy(rows_a, out0.at[sl])
                pltpu.sync_copy(ones, outdeg_b.at[sl])

    return k(src_hbm2d, dst_hbm2d, item_tbl, item0_tbl)


_K_ATT = 1.0
_NREL = 4
_CW = 32


def _tc_colmean_body(it, out):
    out[...] = jnp.broadcast_to(
        jnp.mean(it[...], axis=0, keepdims=True), (8, _DIM))


def _tc_colmean(entity_embed):
    return pl.pallas_call(
        _tc_colmean_body,
        grid=(1,),
        out_shape=jax.ShapeDtypeStruct((8, _DIM), jnp.float32),
        in_specs=[pl.BlockSpec((_N_ITEMS, _DIM), lambda i: (0, 0))],
        out_specs=pl.BlockSpec((8, _DIM), lambda i: (0, 0)),
    )(entity_embed)


def _tc_finish_body(sum1, sum0, dg_a, dg_b, ue, cm, out):
    d = jnp.maximum(dg_a[:, 0:1] + dg_b[:, 0:1], 1.0)
    emb_f = sum1[...] / d
    emb_f0 = sum0[...] / d
    u = ue[...]
    sel = (jax.lax.broadcasted_iota(jnp.int32, (_DIM, _NREL), 0) // _CW
           == jax.lax.broadcasted_iota(jnp.int32, (_DIM, _NREL), 1)
           ).astype(jnp.float32)
    hi = jax.lax.Precision.HIGHEST
    user_att = jax.lax.dot(emb_f * u, sel, precision=hi)
    user_att = jax.nn.relu(user_att) + 1e-10
    mean_att = jax.lax.dot(u * cm[0:1, :], sel, precision=hi)
    mean_att = jax.nn.relu(mean_att) + 1e-08
    att = _K_ATT * jax.nn.relu(user_att / mean_att - 1.0) + 0.01
    score = jnp.tanh(att)
    score_full = jax.lax.dot(score, sel.T, precision=hi)
    out[...] = score_full * emb_f + emb_f0


_UB = 2000


def _tc_finish(sum1, sum0, deg_a, deg_b, user_embed, colmean):
    return pl.pallas_call(
        _tc_finish_body,
        grid=(_N_USERS // _UB,),
        out_shape=jax.ShapeDtypeStruct((_N_USERS, _DIM), jnp.float32),
        in_specs=[
            pl.BlockSpec((_UB, _DIM), lambda i: (i, 0)),
            pl.BlockSpec((_UB, _DIM), lambda i: (i, 0)),
            pl.BlockSpec((_UB, _DEG_W), lambda i: (i, 0)),
            pl.BlockSpec((_UB, _DEG_W), lambda i: (i, 0)),
            pl.BlockSpec((_UB, _DIM), lambda i: (i, 0)),
            pl.BlockSpec((8, _DIM), lambda i: (0, 0)),
        ],
        out_specs=pl.BlockSpec((_UB, _DIM), lambda i: (i, 0)),
    )(sum1, sum0, deg_a, deg_b, user_embed, colmean)


def kernel(edge_index, entity_embed0, entity_embed, user_embed):
    n_extra = _E_PAD - _N_EDGES
    pad_src = jnp.arange(n_extra, dtype=jnp.int32)
    pad_dst = _N_USERS + (jnp.arange(n_extra, dtype=jnp.int32) & 63)
    src2d = jnp.concatenate([edge_index[0], pad_src]).reshape(_NCHUNK, _K)
    dst2d = jnp.concatenate([edge_index[1], pad_dst]).reshape(_NCHUNK, _K)
    colmean = _tc_colmean(entity_embed)
    sum1, sum0, deg_a, deg_b = _sc_segment_sums(src2d, dst2d, entity_embed,
                                                entity_embed0)
    return _tc_finish(sum1, sum0, deg_a, deg_b, user_embed, colmean)


# speedup vs baseline: 10.1867x; 1.0002x over previous
import functools

import jax
import jax.numpy as jnp
from jax import lax
from jax.experimental import pallas as pl
from jax.experimental.pallas import tpu as pltpu
from jax.experimental.pallas import tpu_sc as plsc

_N_USERS = 10000
_N_ITEMS = 10000
_DIM = 128
_N_EDGES = 320000
_K = 128
_NSUB = 16
_E_PAD = 327680
_CPT = _E_PAD // (_K * _NSUB)
_N_PAD = 10240
_RPT = _N_PAD // _NSUB
_ZCH = 128
_DEG_W = 8
_LANES = 16
_IDXB = 16
_NBLK = _CPT // _IDXB
_NCHUNK = _E_PAD // _K


def _sc_segment_sums(src_hbm2d, dst_hbm2d, item_tbl, item0_tbl):
    mesh = plsc.VectorSubcoreMesh(core_axis_name="c", subcore_axis_name="s")
    out_type = (
        jax.ShapeDtypeStruct((_N_PAD, _DIM), jnp.float32),
        jax.ShapeDtypeStruct((_N_PAD, _DIM), jnp.float32),
        jax.ShapeDtypeStruct((_N_PAD, _DEG_W), jnp.float32),
        jax.ShapeDtypeStruct((_N_PAD, _DEG_W), jnp.float32),
    )
    scratch = [
        pltpu.VMEM((_IDXB, _K), jnp.int32),
        pltpu.VMEM((_IDXB, _K), jnp.int32),
        pltpu.VMEM((_IDXB, _K), jnp.int32),
        pltpu.VMEM((_IDXB, _K), jnp.int32),
        pltpu.VMEM((_K, _DIM), jnp.float32),
        pltpu.VMEM((_K, _DIM), jnp.float32),
        pltpu.VMEM((_K, _DEG_W), jnp.float32),
        pltpu.VMEM_SHARED((_N_PAD, _DIM), jnp.float32),
        pltpu.VMEM_SHARED((_N_PAD, _DEG_W), jnp.float32),
        pltpu.SemaphoreType.DMA,
        pltpu.SemaphoreType.DMA,
        pltpu.SemaphoreType.DMA,
        pltpu.SemaphoreType.DMA,
        pltpu.SemaphoreType.DMA,
        pltpu.SemaphoreType.DMA,
    ]

    @functools.partial(pl.kernel, out_type=out_type, mesh=mesh,
                       scratch_types=scratch,
                       compiler_params=pltpu.CompilerParams(
                           use_tc_tiling_on_sc=False))
    def k(src_hbm, dst_hbm, item_hbm, item0_hbm, out1, out0, outdeg_a,
          outdeg_b, src_a, dst_a, src_b, dst_b, rows_a, rows_b, ones,
          acc, dacc, g_a, g_b, s_a, s_b, i_a, i_b):
        c = lax.axis_index("c")
        s = lax.axis_index("s")

        cbase0 = s * _CPT
        pltpu.async_copy(src_hbm.at[pl.ds(cbase0, _IDXB)], src_a, i_a)
        pltpu.async_copy(dst_hbm.at[pl.ds(cbase0, _IDXB)], dst_a, i_a)

        @pl.loop(0, _K)
        def _(i):
            @pl.loop(0, _DIM // _LANES)
            def _(j):
                rows_a[i, pl.ds(j * _LANES, _LANES)] = jnp.zeros(
                    (_LANES,), jnp.float32)

        @pl.loop(0, _RPT // _ZCH)
        def _(b):
            base = s * _RPT + b * _ZCH
            pltpu.sync_copy(rows_a, acc.at[pl.ds(base, _ZCH)])
            pltpu.sync_copy(rows_a.at[pl.ds(0, _ZCH), pl.ds(0, _DEG_W)],
                            dacc.at[pl.ds(base, _ZCH)])

        @pl.loop(0, _K)
        def _(i):
            rows_a[i, pl.ds(0, _LANES)] = jnp.ones((_LANES,), jnp.float32)

        plsc.subcore_barrier()
        pad_sl = pl.ds(_N_PAD - _K, _K)
        pltpu.sync_copy(rows_a.at[pl.ds(0, _K), pl.ds(0, _DEG_W)],
                        dacc.at[pad_sl])
        pltpu.sync_copy(dacc.at[pad_sl], ones)

        plsc.subcore_barrier()

        def fire_gather(sv, i, buf, gsem):
            idx = sv.at[i]

            @pl.when(c == 0)
            def _():
                pltpu.async_copy(item_hbm.at[idx], buf, gsem)

            @pl.when(c == 1)
            def _():
                pltpu.async_copy(item0_hbm.at[idx], buf, gsem)

        def wait_gather(sv, buf, gsem):
            pltpu.make_async_copy(item_hbm.at[sv.at[0]], buf, gsem).wait()

        def fire_scatter(dv, i, buf, ssem, pdeg):
            pltpu.async_copy(buf, acc.at[dv.at[i]], ssem, add=True)

            @pl.when(pdeg)
            def _():
                pltpu.async_copy(ones, dacc.at[dv.at[i]], ssem, add=True)

        def wait_scatter(dv, buf, ssem, pdeg):
            pltpu.make_async_copy(buf, acc.at[dv.at[0]], ssem).wait()

            @pl.when(pdeg)
            def _():
                pltpu.make_async_copy(ones, dacc.at[dv.at[0]],
                                      ssem).wait()

        def stage_idx(b, sbuf, dbuf, isem):
            cbase = s * _CPT + b * _IDXB
            pltpu.async_copy(src_hbm.at[pl.ds(cbase, _IDXB)], sbuf, isem)
            pltpu.async_copy(dst_hbm.at[pl.ds(cbase, _IDXB)], dbuf, isem)

        def wait_idx(sbuf, dbuf, isem):
            pltpu.make_async_copy(src_hbm.at[pl.ds(0, _IDXB)], sbuf,
                                  isem).wait()
            pltpu.make_async_copy(dst_hbm.at[pl.ds(0, _IDXB)], dbuf,
                                  isem).wait()

        bufs = ((rows_a, g_a, s_a), (rows_b, g_b, s_b))
        half = _NBLK // 2

        def process_block(b, sv, dv):
            pdeg = jnp.logical_or(
                jnp.logical_and(c == 0, b < half),
                jnp.logical_and(c == 1, b >= half))
            fire_gather(sv, 0, rows_a, g_a)
            for i in range(_IDXB):
                cur_buf, cur_g, cur_s = bufs[i % 2]
                nxt_buf, nxt_g, nxt_s = bufs[(i + 1) % 2]
                if i + 1 < _IDXB:
                    if i >= 1:
                        wait_scatter(dv, nxt_buf, nxt_s, pdeg)
                    fire_gather(sv, i + 1, nxt_buf, nxt_g)
                wait_gather(sv, cur_buf, cur_g)
                fire_scatter(dv, i, cur_buf, cur_s, pdeg)
            wait_scatter(dv, rows_a, s_a, pdeg)
            wait_scatter(dv, rows_b, s_b, pdeg)

        @pl.loop(0, _NBLK // 2)
        def _(p):
            b0 = 2 * p
            wait_idx(src_a, dst_a, i_a)
            stage_idx(b0 + 1, src_b, dst_b, i_b)
            process_block(b0, src_a, dst_a)
            wait_idx(src_b, dst_b, i_b)

            @pl.when(p + 1 < _NBLK // 2)
            def _():
                stage_idx(b0 + 2, src_a, dst_a, i_a)

            process_block(b0 + 1, src_b, dst_b)

        plsc.subcore_barrier()

        nwb = _RPT // _ZCH
        for b in range(nwb):
            buf, _, ssem = bufs[b % 2]
            sl = pl.ds(s * _RPT + b * _ZCH, _ZCH)
            if b >= 2:
                pltpu.make_async_copy(buf, out1.at[sl], ssem).wait()
            pltpu.sync_copy(acc.at[sl], buf)

            @pl.when(c == 0)
            def _():
                pltpu.async_copy(buf, out1.at[sl], ssem)

            @pl.when(c == 1)
            def _():
                pltpu.async_copy(buf, out0.at[sl], ssem)

        for b in (nwb - 2, nwb - 1):
            buf, _, ssem = bufs[b % 2]
            sl = pl.ds(s * _RPT + b * _ZCH, _ZCH)
            pltpu.make_async_copy(buf, out1.at[sl], ssem).wait()

        @pl.loop(0, nwb)
        def _(b):
            sl = pl.ds(s * _RPT + b * _ZCH, _ZCH)
            pltpu.sync_copy(dacc.at[sl], ones)

            @pl.when(c == 0)
            def _():
                pltpu.sync_copy(ones, outdeg_a.at[sl])

            @pl.when(c == 1)
            def _():
                pltpu.sync_copy(ones, outdeg_b.at[sl])

    return k(src_hbm2d, dst_hbm2d, item_tbl, item0_tbl)


_K_ATT = 1.0
_NREL = 4
_CW = 32


def _tc_finish_body(sum1, sum0, dg_a, dg_b, ue, it, out, cm_ref):
    i = pl.program_id(0)

    @pl.when(i == 0)
    def _():
        cm_ref[...] = jnp.broadcast_to(
            jnp.mean(it[...], axis=0, keepdims=True), (8, _DIM))

    d = jnp.maximum(dg_a[:, 0:1] + dg_b[:, 0:1], 1.0)
    emb_f = sum1[...] / d
    emb_f0 = sum0[...] / d
    u = ue[...]
    sel = (jax.lax.broadcasted_iota(jnp.int32, (_DIM, _NREL), 0) // _CW
           == jax.lax.broadcasted_iota(jnp.int32, (_DIM, _NREL), 1)
           ).astype(jnp.float32)
    hi = jax.lax.Precision.HIGHEST
    user_att = jax.lax.dot(emb_f * u, sel, precision=hi)
    user_att = jax.nn.relu(user_att) + 1e-10
    mean_att = jax.lax.dot(u * cm_ref[0:1, :], sel, precision=hi)
    mean_att = jax.nn.relu(mean_att) + 1e-08
    att = _K_ATT * jax.nn.relu(user_att / mean_att - 1.0) + 0.01
    score = jnp.tanh(att)
    score_full = jax.lax.dot(score, sel.T, precision=hi)
    out[...] = score_full * emb_f + emb_f0


_UB = 2000


def _tc_finish(sum1, sum0, deg_a, deg_b, user_embed, entity_embed):
    return pl.pallas_call(
        _tc_finish_body,
        grid=(_N_USERS // _UB,),
        out_shape=jax.ShapeDtypeStruct((_N_USERS, _DIM), jnp.float32),
        in_specs=[
            pl.BlockSpec((_UB, _DIM), lambda i: (i, 0)),
            pl.BlockSpec((_UB, _DIM), lambda i: (i, 0)),
            pl.BlockSpec((_UB, _DEG_W), lambda i: (i, 0)),
            pl.BlockSpec((_UB, _DEG_W), lambda i: (i, 0)),
            pl.BlockSpec((_UB, _DIM), lambda i: (i, 0)),
            pl.BlockSpec((_N_ITEMS, _DIM), lambda i: (0, 0)),
        ],
        out_specs=pl.BlockSpec((_UB, _DIM), lambda i: (i, 0)),
        scratch_shapes=[pltpu.VMEM((8, _DIM), jnp.float32)],
    )(sum1, sum0, deg_a, deg_b, user_embed, entity_embed)


def kernel(edge_index, entity_embed0, entity_embed, user_embed):
    n_extra = _E_PAD - _N_EDGES
    pad_src = jnp.arange(n_extra, dtype=jnp.int32)
    pad_dst = _N_USERS + (jnp.arange(n_extra, dtype=jnp.int32) & 63)
    src2d = jnp.concatenate([edge_index[0], pad_src]).reshape(_NCHUNK, _K)
    dst2d = jnp.concatenate([edge_index[1], pad_dst]).reshape(_NCHUNK, _K)
    sum1, sum0, deg_a, deg_b = _sc_segment_sums(src2d, dst2d, entity_embed,
                                                entity_embed0)
    return _tc_finish(sum1, sum0, deg_a, deg_b, user_embed, entity_embed)
